# baseline (speedup 1.0000x reference)
import jax
import jax.numpy as jnp
from jax import lax
from jax.experimental import pallas as pl
from jax.experimental.pallas import tpu as pltpu
from jax.experimental.pallas import tpu_sc as plsc

M, D, B = 1000000, 32, 16384
NC, NS = 2, 16
UPT = B // NS
W = 25000
WPC = M // (W * NC)
NDUM = 512
CH = 128
NCH = UPT // CH
ROWS_A = 1568
ROWS_B = W - (NS - 1) * ROWS_A


def _scatter_body(mem_hbm, idx_hbm, val_hbm, out_hbm, idx_v, val_v, lidx_v, win):
    cid = lax.axis_index("c")
    sid = lax.axis_index("s")
    ubase = sid * UPT
    pltpu.sync_copy(idx_hbm.at[pl.ds(ubase, UPT)], idx_v)
    pltpu.sync_copy(val_hbm.at[pl.ds(ubase, UPT)], val_v)

    def window(w, carry):
        row_base = (cid * WPC + w) * W

        @pl.when(sid < NS - 1)
        def _():
            s = sid * ROWS_A
            pltpu.sync_copy(mem_hbm.at[pl.ds(row_base + s, ROWS_A)],
                            win.at[pl.ds(s, ROWS_A)])

        @pl.when(sid == NS - 1)
        def _():
            s = (NS - 1) * ROWS_A
            pltpu.sync_copy(mem_hbm.at[pl.ds(row_base + s, ROWS_B)],
                            win.at[pl.ds(s, ROWS_B)])

        for j in range(UPT // 16):
            iv = idx_v[pl.ds(j * 16, 16)]
            loc = iv - row_base
            inw = (loc >= 0) & (loc < W)
            dum = W + ((j * 16 + lax.iota(jnp.int32, 16)) & (NDUM - 1))
            lidx_v[j // (CH // 16), pl.ds((j % (CH // 16)) * 16, 16)] = (
                jnp.where(inw, loc, dum))

        plsc.subcore_barrier()
        for j in range(NCH):
            pltpu.sync_copy(val_v.at[pl.ds(j * CH, CH)],
                            win.at[lidx_v.at[j]], add=True)
        plsc.subcore_barrier()

        @pl.when(sid < NS - 1)
        def _():
            s = sid * ROWS_A
            pltpu.sync_copy(win.at[pl.ds(s, ROWS_A)],
                            out_hbm.at[pl.ds(row_base + s, ROWS_A)])

        @pl.when(sid == NS - 1)
        def _():
            s = (NS - 1) * ROWS_A
            pltpu.sync_copy(win.at[pl.ds(s, ROWS_B)],
                            out_hbm.at[pl.ds(row_base + s, ROWS_B)])

        plsc.subcore_barrier()
        return carry

    lax.fori_loop(0, WPC, window, 0)


def kernel(mem, idx, val):
    run = pl.kernel(
        _scatter_body,
        out_type=jax.ShapeDtypeStruct((M, D), jnp.float32),
        mesh=plsc.VectorSubcoreMesh(core_axis_name="c", subcore_axis_name="s"),
        compiler_params=pltpu.CompilerParams(use_tc_tiling_on_sc=False),
        scratch_types=[
            pltpu.VMEM((UPT,), jnp.int32),
            pltpu.VMEM((UPT, D), jnp.float32),
            pltpu.VMEM((NCH, CH), jnp.int32),
            pltpu.VMEM_SHARED((W + NDUM, D), jnp.float32),
        ],
    )
    return run(mem, idx.astype(jnp.int32), val)


# speedup vs baseline: 5.6732x; 5.6732x over previous
import jax
import jax.numpy as jnp
from jax import lax
from jax.experimental import pallas as pl
from jax.experimental.pallas import tpu as pltpu
from jax.experimental.pallas import tpu_sc as plsc

M, D, B = 1000000, 32, 16384
NC, NS = 2, 16
UPT = B // NS
CH = 128
NCH = UPT // CH
DPC = D // NC
MAIN = 999936
TAIL = M - MAIN
COLS_A = 62464
COLS_B = MAIN - (NS - 1) * COLS_A


def _scatter_body(mem_t, idx2d, val_t, tail_in, out_t, tail_out,
                  idx_s, val_v, tail_v, row):
    cid = lax.axis_index("c")
    sid = lax.axis_index("s")
    pltpu.sync_copy(idx2d.at[pl.ds(sid * (UPT // CH), UPT // CH)], idx_s)

    def per_dim(k, carry):
        d = cid * DPC + k

        @pl.when(sid < NS - 1)
        def _():
            s = sid * COLS_A
            pltpu.sync_copy(mem_t.at[d, pl.ds(s, COLS_A)],
                            row.at[pl.ds(s, COLS_A)])

        @pl.when(sid == NS - 1)
        def _():
            s = (NS - 1) * COLS_A
            pltpu.sync_copy(mem_t.at[d, pl.ds(s, COLS_B)],
                            row.at[pl.ds(s, COLS_B)])
            pltpu.sync_copy(tail_in.at[d], tail_v)
            pltpu.sync_copy(tail_v.at[pl.ds(0, TAIL)], row.at[pl.ds(MAIN, TAIL)])

        pltpu.sync_copy(val_t.at[d, pl.ds(sid * UPT, UPT)], val_v)
        plsc.subcore_barrier()
        for j in range(NCH):
            pltpu.sync_copy(val_v.at[pl.ds(j * CH, CH)],
                            row.at[idx_s.at[j]], add=True)
        plsc.subcore_barrier()

        @pl.when(sid < NS - 1)
        def _():
            s = sid * COLS_A
            pltpu.sync_copy(row.at[pl.ds(s, COLS_A)],
                            out_t.at[d, pl.ds(s, COLS_A)])

        @pl.when(sid == NS - 1)
        def _():
            s = (NS - 1) * COLS_A
            pltpu.sync_copy(row.at[pl.ds(s, COLS_B)],
                            out_t.at[d, pl.ds(s, COLS_B)])
            pltpu.sync_copy(row.at[pl.ds(MAIN, TAIL)], tail_v.at[pl.ds(0, TAIL)])
            pltpu.sync_copy(tail_v, tail_out.at[d])

        plsc.subcore_barrier()
        return carry

    lax.fori_loop(0, DPC, per_dim, 0)


def kernel(mem, idx, val):
    run = pl.kernel(
        _scatter_body,
        out_type=(jax.ShapeDtypeStruct((D, M), jnp.float32),
                  jax.ShapeDtypeStruct((D, CH), jnp.float32)),
        mesh=plsc.VectorSubcoreMesh(core_axis_name="c", subcore_axis_name="s"),
        scratch_types=[
            pltpu.VMEM((UPT // CH, CH), jnp.int32),
            pltpu.VMEM((UPT,), jnp.float32),
            pltpu.VMEM((CH,), jnp.float32),
            pltpu.VMEM_SHARED((M,), jnp.float32),
        ],
    )
    tail_in = jnp.pad(mem[MAIN:].T, ((0, 0), (0, CH - TAIL)))
    out_t, tail_out = run(mem.T, idx.astype(jnp.int32).reshape(B // CH, CH),
                          val.T, tail_in)
    out = out_t.T
    return lax.dynamic_update_slice(out, tail_out[:, :TAIL].T, (MAIN, 0))


# speedup vs baseline: 9.3283x; 1.6443x over previous
import jax
import jax.numpy as jnp
from jax import lax
from jax.experimental import pallas as pl
from jax.experimental.pallas import tpu as pltpu
from jax.experimental.pallas import tpu_sc as plsc

M, D, B = 1000000, 32, 16384
NC, NS = 2, 16
UPT = B // NS
CH = 128
NCH = UPT // CH
DPC = D // NC
MAIN = 999936
TAIL = M - MAIN
COLS_A = 62464
COLS_B = MAIN - (NS - 1) * COLS_A


def _scatter_body(mem_t, idx2d, val_t, tail_in, out_t, tail_out,
                  idx_s, val_v0, val_v1, tail_v0, tail_v1,
                  row0, row1, lsem, ssem, vsem):
    cid = lax.axis_index("c")
    sid = lax.axis_index("s")
    rows, vals, tails = [row0, row1], [val_v0, val_v1], [tail_v0, tail_v1]
    pltpu.sync_copy(idx2d.at[pl.ds(sid * (UPT // CH), UPT // CH)], idx_s)

    def load_pairs(k):
        d = cid * DPC + k
        buf, tv, vv = rows[k % 2], tails[k % 2], vals[k % 2]
        sa = sid * COLS_A
        sb = (NS - 1) * COLS_A
        main = (mem_t.at[d, pl.ds(sa, COLS_A)], buf.at[pl.ds(sa, COLS_A)])
        last = (mem_t.at[d, pl.ds(sb, COLS_B)], buf.at[pl.ds(sb, COLS_B)])
        return main, last, (tail_in.at[d], tv), (val_t.at[d, pl.ds(sid * UPT, UPT)], vv)

    def store_pairs(k):
        d = cid * DPC + k
        buf, tv = rows[k % 2], tails[k % 2]
        sa = sid * COLS_A
        sb = (NS - 1) * COLS_A
        main = (buf.at[pl.ds(sa, COLS_A)], out_t.at[d, pl.ds(sa, COLS_A)])
        last = (buf.at[pl.ds(sb, COLS_B)], out_t.at[d, pl.ds(sb, COLS_B)])
        return main, last, (tv, tail_out.at[d])

    def issue_load(k):
        main, last, tl, vl = load_pairs(k)
        pltpu.async_copy(vl[0], vl[1], vsem)

        @pl.when(sid < NS - 1)
        def _():
            pltpu.async_copy(main[0], main[1], lsem)

        @pl.when(sid == NS - 1)
        def _():
            pltpu.async_copy(last[0], last[1], lsem)
            pltpu.async_copy(tl[0], tl[1], lsem)

    def wait_load(k):
        main, last, tl, vl = load_pairs(k)
        buf, tv = rows[k % 2], tails[k % 2]
        pltpu.make_async_copy(vl[0], vl[1], vsem).wait()

        @pl.when(sid < NS - 1)
        def _():
            pltpu.make_async_copy(main[0], main[1], lsem).wait()

        @pl.when(sid == NS - 1)
        def _():
            pltpu.make_async_copy(last[0], last[1], lsem).wait()
            pltpu.make_async_copy(tl[0], tl[1], lsem).wait()
            pltpu.sync_copy(tv.at[pl.ds(0, TAIL)], buf.at[pl.ds(MAIN, TAIL)])

    def issue_store(k):
        main, last, ts = store_pairs(k)
        buf, tv = rows[k % 2], tails[k % 2]

        @pl.when(sid < NS - 1)
        def _():
            pltpu.async_copy(main[0], main[1], ssem)

        @pl.when(sid == NS - 1)
        def _():
            pltpu.sync_copy(buf.at[pl.ds(MAIN, TAIL)], tv.at[pl.ds(0, TAIL)])
            pltpu.async_copy(last[0], last[1], ssem)
            pltpu.async_copy(ts[0], ts[1], ssem)

    def wait_store(k):
        main, last, ts = store_pairs(k)

        @pl.when(sid < NS - 1)
        def _():
            pltpu.make_async_copy(main[0], main[1], ssem).wait()

        @pl.when(sid == NS - 1)
        def _():
            pltpu.make_async_copy(last[0], last[1], ssem).wait()
            pltpu.make_async_copy(ts[0], ts[1], ssem).wait()

    issue_load(0)
    for k in range(DPC):
        wait_load(k)
        plsc.subcore_barrier()
        if k + 1 < DPC:
            if k >= 1:
                wait_store(k - 1)
            issue_load(k + 1)
        vv = vals[k % 2]
        for j in range(NCH):
            pltpu.sync_copy(vv.at[pl.ds(j * CH, CH)],
                            rows[k % 2].at[idx_s.at[j]], add=True)
        plsc.subcore_barrier()
        issue_store(k)
    wait_store(DPC - 2)
    wait_store(DPC - 1)


def kernel(mem, idx, val):
    run = pl.kernel(
        _scatter_body,
        out_type=(jax.ShapeDtypeStruct((D, M), jnp.float32),
                  jax.ShapeDtypeStruct((D, CH), jnp.float32)),
        mesh=plsc.VectorSubcoreMesh(core_axis_name="c", subcore_axis_name="s"),
        scratch_types=[
            pltpu.VMEM((UPT // CH, CH), jnp.int32),
            pltpu.VMEM((UPT,), jnp.float32),
            pltpu.VMEM((UPT,), jnp.float32),
            pltpu.VMEM((CH,), jnp.float32),
            pltpu.VMEM((CH,), jnp.float32),
            pltpu.VMEM_SHARED((M,), jnp.float32),
            pltpu.VMEM_SHARED((M,), jnp.float32),
            pltpu.SemaphoreType.DMA,
            pltpu.SemaphoreType.DMA,
            pltpu.SemaphoreType.DMA,
        ],
    )
    tail_in = jnp.pad(mem[MAIN:].T, ((0, 0), (0, CH - TAIL)))
    out_t, tail_out = run(mem.T, idx.astype(jnp.int32).reshape(B // CH, CH),
                          val.T, tail_in)
    out = out_t.T
    return lax.dynamic_update_slice(out, tail_out[:, :TAIL].T, (MAIN, 0))


# speedup vs baseline: 9.8202x; 1.0527x over previous
import jax
import jax.numpy as jnp
from jax import lax
from jax.experimental import pallas as pl
from jax.experimental.pallas import tpu as pltpu
from jax.experimental.pallas import tpu_sc as plsc

M, D, B = 1000000, 32, 16384
NC, NS = 2, 16
UPT = B // NS
CH = 128
NCH = UPT // CH
DPC = D // NC
MAIN = 999936
TAIL = M - MAIN
COLS_A = 62464
COLS_B = MAIN - (NS - 1) * COLS_A


def _scatter_body(mem_t, idx2d, val_t, tail_in, out_t, tail_out,
                  idx_s, val_v0, val_v1, tail_v0, tail_v1,
                  row0, row1, lsem, ssem, vsem, csem):
    cid = lax.axis_index("c")
    sid = lax.axis_index("s")
    rows, vals, tails = [row0, row1], [val_v0, val_v1], [tail_v0, tail_v1]
    pltpu.sync_copy(idx2d.at[pl.ds(sid * (UPT // CH), UPT // CH)], idx_s)

    def load_pairs(k):
        d = cid * DPC + k
        buf, tv, vv = rows[k % 2], tails[k % 2], vals[k % 2]
        sa = sid * COLS_A
        sb = (NS - 1) * COLS_A
        main = (mem_t.at[d, pl.ds(sa, COLS_A)], buf.at[pl.ds(sa, COLS_A)])
        last = (mem_t.at[d, pl.ds(sb, COLS_B)], buf.at[pl.ds(sb, COLS_B)])
        return main, last, (tail_in.at[d], tv), (val_t.at[d, pl.ds(sid * UPT, UPT)], vv)

    def store_pairs(k):
        d = cid * DPC + k
        buf, tv = rows[k % 2], tails[k % 2]
        sa = sid * COLS_A
        sb = (NS - 1) * COLS_A
        main = (buf.at[pl.ds(sa, COLS_A)], out_t.at[d, pl.ds(sa, COLS_A)])
        last = (buf.at[pl.ds(sb, COLS_B)], out_t.at[d, pl.ds(sb, COLS_B)])
        return main, last, (tv, tail_out.at[d])

    def issue_load(k):
        main, last, tl, vl = load_pairs(k)
        pltpu.async_copy(vl[0], vl[1], vsem)

        @pl.when(sid < NS - 1)
        def _():
            pltpu.async_copy(main[0], main[1], lsem)

        @pl.when(sid == NS - 1)
        def _():
            pltpu.async_copy(last[0], last[1], lsem)
            pltpu.async_copy(tl[0], tl[1], lsem)

    def wait_load(k):
        main, last, tl, vl = load_pairs(k)
        buf, tv = rows[k % 2], tails[k % 2]
        pltpu.make_async_copy(vl[0], vl[1], vsem).wait()

        @pl.when(sid < NS - 1)
        def _():
            pltpu.make_async_copy(main[0], main[1], lsem).wait()

        @pl.when(sid == NS - 1)
        def _():
            pltpu.make_async_copy(last[0], last[1], lsem).wait()
            pltpu.make_async_copy(tl[0], tl[1], lsem).wait()
            pltpu.sync_copy(tv.at[pl.ds(0, TAIL)], buf.at[pl.ds(MAIN, TAIL)])

    def issue_store(k):
        main, last, ts = store_pairs(k)
        buf, tv = rows[k % 2], tails[k % 2]

        @pl.when(sid < NS - 1)
        def _():
            pltpu.async_copy(main[0], main[1], ssem)

        @pl.when(sid == NS - 1)
        def _():
            pltpu.sync_copy(buf.at[pl.ds(MAIN, TAIL)], tv.at[pl.ds(0, TAIL)])
            pltpu.async_copy(last[0], last[1], ssem)
            pltpu.async_copy(ts[0], ts[1], ssem)

    def wait_store(k):
        main, last, ts = store_pairs(k)

        @pl.when(sid < NS - 1)
        def _():
            pltpu.make_async_copy(main[0], main[1], ssem).wait()

        @pl.when(sid == NS - 1)
        def _():
            pltpu.make_async_copy(last[0], last[1], ssem).wait()
            pltpu.make_async_copy(ts[0], ts[1], ssem).wait()

    issue_load(0)
    for k in range(DPC):
        wait_load(k)
        plsc.subcore_barrier()
        if k + 1 < DPC:
            if k >= 1:
                wait_store(k - 1)
            issue_load(k + 1)
        vv = vals[k % 2]
        for j in range(NCH):
            pltpu.async_copy(vv.at[pl.ds(j * CH, CH)],
                             rows[k % 2].at[idx_s.at[j]], csem, add=True)
        for j in range(NCH):
            pltpu.make_async_copy(vv.at[pl.ds(j * CH, CH)],
                                  rows[k % 2].at[idx_s.at[j]], csem).wait()
        plsc.subcore_barrier()
        issue_store(k)
    wait_store(DPC - 2)
    wait_store(DPC - 1)


def kernel(mem, idx, val):
    run = pl.kernel(
        _scatter_body,
        out_type=(jax.ShapeDtypeStruct((D, M), jnp.float32),
                  jax.ShapeDtypeStruct((D, CH), jnp.float32)),
        mesh=plsc.VectorSubcoreMesh(core_axis_name="c", subcore_axis_name="s"),
        scratch_types=[
            pltpu.VMEM((UPT // CH, CH), jnp.int32),
            pltpu.VMEM((UPT,), jnp.float32),
            pltpu.VMEM((UPT,), jnp.float32),
            pltpu.VMEM((CH,), jnp.float32),
            pltpu.VMEM((CH,), jnp.float32),
            pltpu.VMEM_SHARED((M,), jnp.float32),
            pltpu.VMEM_SHARED((M,), jnp.float32),
            pltpu.SemaphoreType.DMA,
            pltpu.SemaphoreType.DMA,
            pltpu.SemaphoreType.DMA,
            pltpu.SemaphoreType.DMA,
        ],
    )
    tail_in = jnp.pad(mem[MAIN:].T, ((0, 0), (0, CH - TAIL)))
    out_t, tail_out = run(mem.T, idx.astype(jnp.int32).reshape(B // CH, CH),
                          val.T, tail_in)
    out = out_t.T
    return lax.dynamic_update_slice(out, tail_out[:, :TAIL].T, (MAIN, 0))
